# Initial kernel scaffold; baseline (speedup 1.0000x reference)
import jax
import jax.numpy as jnp
from jax.experimental import pallas as pl


def kernel(title_idx, text_tokens, title_table, text_table):
    raise NotImplementedError("write your pallas kernel here")


# speedup vs baseline: 8.8007x; 8.8007x over previous
import functools

import jax
import jax.numpy as jnp
from jax import lax
from jax.experimental import pallas as pl
from jax.experimental.pallas import tpu as pltpu
from jax.experimental.pallas import tpu_sc as plsc

D = 32
B = 16384
SEQ = 20
NC, NS = 2, 16
NW = NC * NS
BPW = B // NW
CH = 4
TPC = CH * SEQ
NCHUNK = BPW // CH
L = 16


def _body(title_idx_hbm, tok_hbm, title_tab_hbm, text_tab_hbm, out_hbm,
          title_idx_v, tok_idx_v, title_rows_v, tok_rows_v, out_v, sem):
    wid = lax.axis_index("s") * NC + lax.axis_index("c")
    base = wid * BPW

    pltpu.sync_copy(title_idx_hbm.at[pl.ds(base, BPW)], title_idx_v)
    pltpu.sync_copy(tok_hbm.at[pl.ds(base * SEQ, BPW * SEQ)], tok_idx_v)

    for j in range(BPW // 128):
        pltpu.async_copy(
            title_tab_hbm.at[title_idx_v.at[pl.ds(j * 128, 128)]],
            title_rows_v.at[pl.ds(j * 128, 128)], sem).wait()

    @pl.loop(0, NCHUNK)
    def chunk(c):
        pltpu.async_copy(
            text_tab_hbm.at[tok_idx_v.at[pl.ds(c * TPC, TPC)]],
            tok_rows_v, sem).wait()
        ms = []
        for k in range(TPC // L):
            tv = tok_idx_v[pl.ds(c * TPC + k * L, L)]
            ms.append(jnp.minimum(tv, 1).astype(jnp.float32))
        for r in range(CH):
            acc0 = jnp.zeros((L,), jnp.float32)
            acc1 = jnp.zeros((L,), jnp.float32)
            cnt = jnp.float32(0.0)
            for t in range(SEQ):
                f = r * SEQ + t
                m = ms[f // L][f % L]
                acc0 = acc0 + tok_rows_v[f, 0:L] * m
                acc1 = acc1 + tok_rows_v[f, L:2 * L] * m
                cnt = cnt + m
            den_v = jnp.maximum(jnp.full((L,), cnt, jnp.float32),
                                jnp.float32(1e-9))
            inv_v = jnp.float32(1.0) / den_v
            row = c * CH + r
            out_v[row, 2 * L:3 * L] = acc0 * inv_v
            out_v[row, 3 * L:4 * L] = acc1 * inv_v
            out_v[row, 0:L] = title_rows_v[row, 0:L]
            out_v[row, L:2 * L] = title_rows_v[row, L:2 * L]

    pltpu.sync_copy(out_v, out_hbm.at[pl.ds(base, BPW)])


_sc_call = pl.kernel(
    _body,
    out_type=jax.ShapeDtypeStruct((B, 2 * D), jnp.float32),
    mesh=plsc.VectorSubcoreMesh(
        core_axis_name="c", subcore_axis_name="s",
        num_cores=NC, num_subcores=NS),
    scratch_types=[
        pltpu.VMEM((BPW,), jnp.int32),
        pltpu.VMEM((BPW * SEQ,), jnp.int32),
        pltpu.VMEM((BPW, D), jnp.float32),
        pltpu.VMEM((TPC, D), jnp.float32),
        pltpu.VMEM((BPW, 2 * D), jnp.float32),
        pltpu.SemaphoreType.DMA,
    ],
    compiler_params=pltpu.CompilerParams(use_tc_tiling_on_sc=False),
)


@jax.jit
def kernel(title_idx, text_tokens, title_table, text_table):
    ti = title_idx.astype(jnp.int32)
    tok = text_tokens.astype(jnp.int32).reshape(-1)
    return _sc_call(ti, tok, title_table, text_table)


# speedup vs baseline: 11.9365x; 1.3563x over previous
import functools

import jax
import jax.numpy as jnp
from jax import lax
from jax.experimental import pallas as pl
from jax.experimental.pallas import tpu as pltpu
from jax.experimental.pallas import tpu_sc as plsc

D = 32
B = 16384
SEQ = 20
NC, NS = 2, 16
NW = NC * NS
BPW = B // NW
CH = 4
TPC = CH * SEQ
NCHUNK = BPW // CH
NBUF = 2
L = 16
TCHUNK = 128


def _body(title_idx_hbm, tok_hbm, title_tab_hbm, text_tab_hbm, out_hbm,
          title_idx_v, tok_idx_v, title_rows_v, tok_rows_v, out_v,
          sem_a, sem_b, sem_title):
    wid = lax.axis_index("s") * NC + lax.axis_index("c")
    base = wid * BPW
    sems = [sem_a, sem_b]

    pltpu.sync_copy(title_idx_hbm.at[pl.ds(base, BPW)], title_idx_v)

    for j in range(BPW // TCHUNK):
        pltpu.async_copy(
            title_tab_hbm.at[title_idx_v.at[pl.ds(j * TCHUNK, TCHUNK)]],
            title_rows_v.at[pl.ds(j * TCHUNK, TCHUNK)], sem_title)

    pltpu.sync_copy(tok_hbm.at[pl.ds(base * SEQ, BPW * SEQ)], tok_idx_v)

    def issue(b, c):
        pltpu.async_copy(
            text_tab_hbm.at[tok_idx_v.at[pl.ds(c * TPC, TPC)]],
            tok_rows_v.at[b], sems[b])

    def drain(b):
        pltpu.make_async_copy(
            text_tab_hbm.at[pl.ds(0, TPC)], tok_rows_v.at[b], sems[b]).wait()

    for b in range(NBUF):
        issue(b, b)

    for j in range(BPW // TCHUNK):
        pltpu.make_async_copy(
            title_tab_hbm.at[pl.ds(0, TCHUNK)],
            title_rows_v.at[pl.ds(j * TCHUNK, TCHUNK)], sem_title).wait()

    @pl.loop(0, NCHUNK, step=NBUF)
    def outer(c0):
        for b in range(NBUF):
            c = c0 + b
            drain(b)
            ms = []
            for k in range(TPC // L):
                tv = tok_idx_v[pl.ds(c * TPC + k * L, L)]
                ms.append(jnp.minimum(tv, 1).astype(jnp.float32))
            for r in range(CH):
                acc0 = jnp.zeros((L,), jnp.float32)
                acc1 = jnp.zeros((L,), jnp.float32)
                cnt = jnp.float32(0.0)
                for t in range(SEQ):
                    f = r * SEQ + t
                    m = ms[f // L][f % L]
                    acc0 = acc0 + tok_rows_v[b, f, 0:L] * m
                    acc1 = acc1 + tok_rows_v[b, f, L:2 * L] * m
                    cnt = cnt + m
                den_v = jnp.maximum(jnp.full((L,), cnt, jnp.float32),
                                    jnp.float32(1e-9))
                inv_v = jnp.float32(1.0) / den_v
                row = c * CH + r
                out_v[row, 2 * L:3 * L] = acc0 * inv_v
                out_v[row, 3 * L:4 * L] = acc1 * inv_v
                out_v[row, 0:L] = title_rows_v[row, 0:L]
                out_v[row, L:2 * L] = title_rows_v[row, L:2 * L]
            nxt = c + NBUF
            @pl.when(nxt < NCHUNK)
            def _():
                issue(b, nxt)

    pltpu.sync_copy(out_v, out_hbm.at[pl.ds(base, BPW)])


_sc_call = pl.kernel(
    _body,
    out_type=jax.ShapeDtypeStruct((B, 2 * D), jnp.float32),
    mesh=plsc.VectorSubcoreMesh(
        core_axis_name="c", subcore_axis_name="s",
        num_cores=NC, num_subcores=NS),
    scratch_types=[
        pltpu.VMEM((BPW,), jnp.int32),
        pltpu.VMEM((BPW * SEQ,), jnp.int32),
        pltpu.VMEM((BPW, D), jnp.float32),
        pltpu.VMEM((NBUF, TPC, D), jnp.float32),
        pltpu.VMEM((BPW, 2 * D), jnp.float32),
        pltpu.SemaphoreType.DMA,
        pltpu.SemaphoreType.DMA,
        pltpu.SemaphoreType.DMA,
    ],
    compiler_params=pltpu.CompilerParams(use_tc_tiling_on_sc=False),
)


@jax.jit
def kernel(title_idx, text_tokens, title_table, text_table):
    ti = title_idx.astype(jnp.int32)
    tok = text_tokens.astype(jnp.int32).reshape(-1)
    return _sc_call(ti, tok, title_table, text_table)
